# baseline (speedup 1.0000x reference)
import functools

import jax
import jax.numpy as jnp
from jax import lax
from jax.experimental import pallas as pl
from jax.experimental.pallas import tpu as pltpu
from jax.experimental.pallas import tpu_sc as plsc

B = 16384
D = 26
M = 64
R = 16
RR = R * R
H = 52
NMID = D - 2
BBLK1 = 2048
BBLK2 = 512
NW = 32
BPW = B // NW
CH = 128


def _k1(pT_ref, p_ref, nodesT_ref, nodes_ref, cf_ref, cl_ref,
        w1t_ref, b1r_ref, w2t_ref, b2r_ref, w3c_ref, b3_ref,
        idx_ref, v0_ref, last_ref, nn_ref):
    f32 = jnp.float32
    dn = (((1,), (0,)), ((), ()))
    iota_col = lax.broadcasted_iota(jnp.int32, (M, 1), 0)
    rows = []
    for d in range(1, D - 1):
        dist = jnp.abs(pT_ref[d:d + 1, :] - nodesT_ref[:, d:d + 1])
        minv = jnp.min(dist, axis=0, keepdims=True)
        prio = jnp.where(dist == minv, iota_col, jnp.int32(M))
        amin = jnp.min(prio, axis=0, keepdims=True)
        rows.append(amin + jnp.int32((d - 1) * M))
    idx_ref[...] = jnp.concatenate(rows, axis=0)

    iota_row = lax.broadcasted_iota(jnp.int32, (1, M), 1)

    def onehot_bs(d):
        dist = jnp.abs(p_ref[:, d:d + 1] - nodes_ref[d:d + 1, :])
        minv = jnp.min(dist, axis=1, keepdims=True)
        prio = jnp.where(dist == minv, iota_row, jnp.int32(M))
        amin = jnp.min(prio, axis=1, keepdims=True)
        return (iota_row == amin).astype(f32)

    v0_ref[...] = lax.dot_general(onehot_bs(0), cf_ref[...], dn,
                                  preferred_element_type=f32)
    last_ref[...] = lax.dot_general(onehot_bs(D - 1), cl_ref[...], dn,
                                    preferred_element_type=f32)

    p = p_ref[...]
    h1 = lax.dot_general(p, w1t_ref[...], dn, preferred_element_type=f32)
    h1 = jnp.maximum(h1 + b1r_ref[...], 0.0)
    h2 = lax.dot_general(h1, w2t_ref[...], dn, preferred_element_type=f32)
    h2 = jnp.maximum(h2 + b2r_ref[...], 0.0)
    nn = lax.dot_general(h2, w3c_ref[...], dn, preferred_element_type=f32)
    nn_ref[...] = nn[:, 0] + b3_ref[0]


def _run_k1(pT, p, nodesT, nodes, cf, cl, w1t, b1r, w2t, b2r, w3c, b3):
    grid = (B // BBLK1,)
    whole = lambda shape: pl.BlockSpec(shape, lambda i: tuple(0 for _ in shape))
    return pl.pallas_call(
        _k1,
        grid=grid,
        in_specs=[
            pl.BlockSpec((D, BBLK1), lambda i: (0, i)),
            pl.BlockSpec((BBLK1, D), lambda i: (i, 0)),
            whole((M, D)),
            whole((D, M)),
            whole((M, R)),
            whole((M, R)),
            whole((D, H)),
            whole((1, H)),
            whole((H, H)),
            whole((1, H)),
            whole((H, 1)),
            whole((1,)),
        ],
        out_specs=[
            pl.BlockSpec((NMID, BBLK1), lambda i: (0, i)),
            pl.BlockSpec((BBLK1, R), lambda i: (i, 0)),
            pl.BlockSpec((BBLK1, R), lambda i: (i, 0)),
            pl.BlockSpec((BBLK1,), lambda i: (i,)),
        ],
        out_shape=[
            jax.ShapeDtypeStruct((NMID, B), jnp.int32),
            jax.ShapeDtypeStruct((B, R), jnp.float32),
            jax.ShapeDtypeStruct((B, R), jnp.float32),
            jax.ShapeDtypeStruct((B,), jnp.float32),
        ],
    )(pT, p, nodesT, nodes, cf, cl, w1t, b1r, w2t, b2r, w3c, b3)


def _sc_gather_body(table_hbm, idxg_hbm, out_hbm, idx_v, rows_v, sem):
    wid = lax.axis_index("s") * 2 + lax.axis_index("c")
    base = wid * BPW

    def body(d, carry):
        for c in range(BPW // CH):
            off = base + c * CH
            pltpu.sync_copy(idxg_hbm.at[d, pl.ds(off, CH)], idx_v)
            pltpu.async_copy(table_hbm.at[idx_v], rows_v, sem).wait()
            pltpu.sync_copy(rows_v, out_hbm.at[d, pl.ds(off, CH)])
        return carry

    lax.fori_loop(0, NMID, body, 0)


def _run_sc_gather(table, idxg):
    mesh = plsc.VectorSubcoreMesh(core_axis_name="c", subcore_axis_name="s")
    fn = functools.partial(
        pl.kernel,
        mesh=mesh,
        out_type=jax.ShapeDtypeStruct((NMID, B, RR), jnp.float32),
        scratch_types=[
            pltpu.VMEM((CH,), jnp.int32),
            pltpu.VMEM((CH, RR), jnp.float32),
            pltpu.SemaphoreType.DMA,
        ],
    )(_sc_gather_body)
    return fn(table, idxg)


def _k2(S_ref, v0_ref, last_ref, nn_ref, out_ref):
    v = v0_ref[...]
    for d in range(NMID):
        Sd = S_ref[d]
        acc = v[:, 0:1] * Sd[:, 0:R]
        for r in range(1, R):
            acc = acc + v[:, r:r + 1] * Sd[:, r * R:(r + 1) * R]
        v = acc
    tt = jnp.sum(v * last_ref[...], axis=1)
    out_ref[...] = tt + nn_ref[...]


def _run_k2(S, v0, last, nn):
    grid = (B // BBLK2,)
    return pl.pallas_call(
        _k2,
        grid=grid,
        in_specs=[
            pl.BlockSpec((NMID, BBLK2, RR), lambda i: (0, i, 0)),
            pl.BlockSpec((BBLK2, R), lambda i: (i, 0)),
            pl.BlockSpec((BBLK2, R), lambda i: (i, 0)),
            pl.BlockSpec((BBLK2,), lambda i: (i,)),
        ],
        out_specs=pl.BlockSpec((BBLK2,), lambda i: (i,)),
        out_shape=jax.ShapeDtypeStruct((B,), jnp.float32),
    )(S, v0, last, nn)


@jax.jit
def kernel(points, core_first, cores_mid, core_last, nodes, W1, b1, W2, b2, W3, b3):
    pT = points.T
    nodesT = nodes.T
    cf = core_first[0]
    cl = core_last[:, :, 0].T
    table = jnp.transpose(cores_mid, (0, 2, 1, 3)).reshape(NMID * M, RR)
    idxg, v0, last, nn = _run_k1(pT, points, nodesT, nodes, cf, cl,
                                 W1.T, b1[None, :], W2.T, b2[None, :],
                                 W3.T, b3)
    S = _run_sc_gather(table, idxg)
    return _run_k2(S, v0, last, nn)


# speedup vs baseline: 1.0238x; 1.0238x over previous
import functools

import jax
import jax.numpy as jnp
from jax import lax
from jax.experimental import pallas as pl
from jax.experimental.pallas import tpu as pltpu
from jax.experimental.pallas import tpu_sc as plsc

B = 16384
D = 26
M = 64
R = 16
RR = R * R
H = 52
NMID = D - 2
BBLK1 = 2048
BBLK2 = 512
NW = 32
BPW = B // NW
CH = 128


def _k1(pT_ref, p_ref, nodesT_ref, nodes_ref, cf_ref, cl_ref,
        w1t_ref, b1r_ref, w2t_ref, b2r_ref, w3c_ref, b3_ref,
        idx_ref, v0_ref, last_ref, nn_ref):
    f32 = jnp.float32
    dn = (((1,), (0,)), ((), ()))
    iota_col = lax.broadcasted_iota(jnp.int32, (M, 1), 0)
    rows = []
    for d in range(1, D - 1):
        dist = jnp.abs(pT_ref[d:d + 1, :] - nodesT_ref[:, d:d + 1])
        minv = jnp.min(dist, axis=0, keepdims=True)
        prio = jnp.where(dist == minv, iota_col, jnp.int32(M))
        amin = jnp.min(prio, axis=0, keepdims=True)
        rows.append(amin + jnp.int32((d - 1) * M))
    idx_ref[...] = jnp.concatenate(rows, axis=0)

    iota_row = lax.broadcasted_iota(jnp.int32, (1, M), 1)

    def onehot_bs(d):
        dist = jnp.abs(p_ref[:, d:d + 1] - nodes_ref[d:d + 1, :])
        minv = jnp.min(dist, axis=1, keepdims=True)
        prio = jnp.where(dist == minv, iota_row, jnp.int32(M))
        amin = jnp.min(prio, axis=1, keepdims=True)
        return (iota_row == amin).astype(f32)

    v0_ref[...] = lax.dot_general(onehot_bs(0), cf_ref[...], dn,
                                  preferred_element_type=f32)
    last_ref[...] = lax.dot_general(onehot_bs(D - 1), cl_ref[...], dn,
                                    preferred_element_type=f32)

    p = p_ref[...]
    h1 = lax.dot_general(p, w1t_ref[...], dn, preferred_element_type=f32)
    h1 = jnp.maximum(h1 + b1r_ref[...], 0.0)
    h2 = lax.dot_general(h1, w2t_ref[...], dn, preferred_element_type=f32)
    h2 = jnp.maximum(h2 + b2r_ref[...], 0.0)
    nn = lax.dot_general(h2, w3c_ref[...], dn, preferred_element_type=f32)
    nn_ref[...] = nn[:, 0] + b3_ref[0]


def _run_k1(pT, p, nodesT, nodes, cf, cl, w1t, b1r, w2t, b2r, w3c, b3):
    grid = (B // BBLK1,)
    whole = lambda shape: pl.BlockSpec(shape, lambda i: tuple(0 for _ in shape))
    return pl.pallas_call(
        _k1,
        grid=grid,
        in_specs=[
            pl.BlockSpec((D, BBLK1), lambda i: (0, i)),
            pl.BlockSpec((BBLK1, D), lambda i: (i, 0)),
            whole((M, D)),
            whole((D, M)),
            whole((M, R)),
            whole((M, R)),
            whole((D, H)),
            whole((1, H)),
            whole((H, H)),
            whole((1, H)),
            whole((H, 1)),
            whole((1,)),
        ],
        out_specs=[
            pl.BlockSpec((NMID, BBLK1), lambda i: (0, i)),
            pl.BlockSpec((BBLK1, R), lambda i: (i, 0)),
            pl.BlockSpec((BBLK1, R), lambda i: (i, 0)),
            pl.BlockSpec((BBLK1,), lambda i: (i,)),
        ],
        out_shape=[
            jax.ShapeDtypeStruct((NMID, B), jnp.int32),
            jax.ShapeDtypeStruct((B, R), jnp.float32),
            jax.ShapeDtypeStruct((B, R), jnp.float32),
            jax.ShapeDtypeStruct((B,), jnp.float32),
        ],
    )(pT, p, nodesT, nodes, cf, cl, w1t, b1r, w2t, b2r, w3c, b3)


NCH = BPW // CH


def _sc_gather_body(table_hbm, idxg_hbm, out_hbm, idx_v, rows0, rows1,
                    g0, g1, s0, s1):
    wid = lax.axis_index("s") * 2 + lax.axis_index("c")
    base = wid * BPW
    rows = (rows0, rows1)
    gsem = (g0, g1)
    ssem = (s0, s1)

    def body(d, carry):
        pltpu.sync_copy(idxg_hbm.at[d, wid], idx_v)
        hg = [None, None]
        hs = [None, None]
        for b in range(2):
            hg[b] = pltpu.async_copy(table_hbm.at[idx_v.at[b]], rows[b],
                                     gsem[b])
        for c in range(NCH):
            b = c % 2
            hg[b].wait()
            hs[b] = pltpu.async_copy(rows[b],
                                     out_hbm.at[d, pl.ds(base + c * CH, CH)],
                                     ssem[b])
            if c + 2 < NCH:
                hs[b].wait()
                hg[b] = pltpu.async_copy(table_hbm.at[idx_v.at[c + 2]],
                                         rows[b], gsem[b])
        hs[0].wait()
        hs[1].wait()
        return carry

    lax.fori_loop(0, NMID, body, 0)


def _run_sc_gather(table, idxg4):
    mesh = plsc.VectorSubcoreMesh(core_axis_name="c", subcore_axis_name="s")
    fn = functools.partial(
        pl.kernel,
        mesh=mesh,
        out_type=jax.ShapeDtypeStruct((NMID, B, RR), jnp.float32),
        scratch_types=[
            pltpu.VMEM((NCH, CH), jnp.int32),
            pltpu.VMEM((CH, RR), jnp.float32),
            pltpu.VMEM((CH, RR), jnp.float32),
            pltpu.SemaphoreType.DMA,
            pltpu.SemaphoreType.DMA,
            pltpu.SemaphoreType.DMA,
            pltpu.SemaphoreType.DMA,
        ],
    )(_sc_gather_body)
    return fn(table, idxg4)


def _k2(S_ref, v0_ref, last_ref, nn_ref, out_ref):
    v = v0_ref[...]
    for d in range(NMID):
        Sd = S_ref[d]
        acc = v[:, 0:1] * Sd[:, 0:R]
        for r in range(1, R):
            acc = acc + v[:, r:r + 1] * Sd[:, r * R:(r + 1) * R]
        v = acc
    tt = jnp.sum(v * last_ref[...], axis=1)
    out_ref[...] = tt + nn_ref[...]


def _run_k2(S, v0, last, nn):
    grid = (B // BBLK2,)
    return pl.pallas_call(
        _k2,
        grid=grid,
        in_specs=[
            pl.BlockSpec((NMID, BBLK2, RR), lambda i: (0, i, 0)),
            pl.BlockSpec((BBLK2, R), lambda i: (i, 0)),
            pl.BlockSpec((BBLK2, R), lambda i: (i, 0)),
            pl.BlockSpec((BBLK2,), lambda i: (i,)),
        ],
        out_specs=pl.BlockSpec((BBLK2,), lambda i: (i,)),
        out_shape=jax.ShapeDtypeStruct((B,), jnp.float32),
    )(S, v0, last, nn)


@jax.jit
def kernel(points, core_first, cores_mid, core_last, nodes, W1, b1, W2, b2, W3, b3):
    pT = points.T
    nodesT = nodes.T
    cf = core_first[0]
    cl = core_last[:, :, 0].T
    table = jnp.transpose(cores_mid, (0, 2, 1, 3)).reshape(NMID * M, RR)
    idxg, v0, last, nn = _run_k1(pT, points, nodesT, nodes, cf, cl,
                                 W1.T, b1[None, :], W2.T, b2[None, :],
                                 W3.T, b3)
    S = _run_sc_gather(table, idxg.reshape(NMID, NW, NCH, CH))
    return _run_k2(S, v0, last, nn)


# speedup vs baseline: 17.8340x; 17.4201x over previous
import functools

import jax
import jax.numpy as jnp
from jax import lax
from jax.experimental import pallas as pl
from jax.experimental.pallas import tpu as pltpu
from jax.experimental.pallas import tpu_sc as plsc

B = 16384
D = 26
M = 64
R = 16
RR = R * R
H = 52
NMID = D - 2
NW = 32
BSC = 1024
BPW = BSC // NW
HALF = NMID // 2
BMAIN = B - BSC
BBLK = 1024


def _first_argmin_lanes(x_row, nodes_col, iota_col):
    dist = jnp.abs(x_row - nodes_col)
    minv = jnp.min(dist, axis=0, keepdims=True)
    prio = jnp.where(dist == minv, iota_col, jnp.int32(M))
    return jnp.min(prio, axis=0, keepdims=True)


def _mlp_lanes(pT, w1_ref, b1_ref, w2_ref, b2_ref, w3_ref, b3_ref):
    f32 = jnp.float32
    dn = (((1,), (0,)), ((), ()))
    h1 = lax.dot_general(w1_ref[...], pT, dn, preferred_element_type=f32)
    h1 = jnp.maximum(h1 + b1_ref[...], 0.0)
    h2 = lax.dot_general(w2_ref[...], h1, dn, preferred_element_type=f32)
    h2 = jnp.maximum(h2 + b2_ref[...], 0.0)
    nn = lax.dot_general(w3_ref[...], h2, dn, preferred_element_type=f32)
    return nn + b3_ref[0]


def _k1_body(pT_ref, nodesT_ref, cfT_ref, clT_ref,
             w1_ref, b1_ref, w2_ref, b2_ref, w3_ref, b3_ref,
             idx_ref, v0_ref, last_ref, nn_ref):
    f32 = jnp.float32
    dn = (((1,), (0,)), ((), ()))
    iota_col = lax.broadcasted_iota(jnp.int32, (M, 1), 0)
    rows = []
    for d in range(1, D - 1):
        amin = _first_argmin_lanes(pT_ref[d:d + 1, :], nodesT_ref[:, d:d + 1],
                                   iota_col)
        rows.append(amin + jnp.int32((d - 1) * M))
    idx_ref[...] = jnp.concatenate(rows, axis=0)

    def onehot(d):
        amin = _first_argmin_lanes(pT_ref[d:d + 1, :], nodesT_ref[:, d:d + 1],
                                   iota_col)
        return (iota_col == amin).astype(f32)

    v0_ref[...] = lax.dot_general(cfT_ref[...], onehot(0), dn,
                                  preferred_element_type=f32)
    last_ref[...] = lax.dot_general(clT_ref[...], onehot(D - 1), dn,
                                    preferred_element_type=f32)
    nn_ref[...] = _mlp_lanes(pT_ref[...], w1_ref, b1_ref, w2_ref, b2_ref,
                             w3_ref, b3_ref)[0, :]


def _run_k1(pT_sc, nodesT, cfT, clT, w1, b1c, w2, b2c, w3, b3):
    whole = lambda shape: pl.BlockSpec(shape, lambda: tuple(0 for _ in shape))
    return pl.pallas_call(
        _k1_body,
        in_specs=[whole((D, BSC)), whole((M, D)), whole((R, M)), whole((R, M)),
                  whole((H, D)), whole((H, 1)), whole((H, H)), whole((H, 1)),
                  whole((1, H)), whole((1,))],
        out_specs=[whole((NMID, BSC)), whole((R, BSC)), whole((R, BSC)),
                   whole((BSC,))],
        out_shape=[
            jax.ShapeDtypeStruct((NMID, BSC), jnp.int32),
            jax.ShapeDtypeStruct((R, BSC), jnp.float32),
            jax.ShapeDtypeStruct((R, BSC), jnp.float32),
            jax.ShapeDtypeStruct((BSC,), jnp.float32),
        ],
    )(pT_sc, nodesT, cfT, clT, w1, b1c, w2, b2c, w3, b3)


def _sc_body(table_hbm, idxw_hbm, out_hbm, idx_all, bufs, gsem, ssem):
    wid = lax.axis_index("s") * 2 + lax.axis_index("c")
    pltpu.sync_copy(idxw_hbm.at[wid], idx_all)
    for half in range(2):
        hg = []
        for t in range(HALF):
            d = half * HALF + t
            hg.append(pltpu.async_copy(table_hbm.at[idx_all.at[d]],
                                       bufs.at[t], gsem))
        for h in hg:
            h.wait()
        hs = []
        for t in range(HALF):
            d = half * HALF + t
            hs.append(pltpu.async_copy(
                bufs.at[t], out_hbm.at[d, pl.ds(wid * BPW, BPW)], ssem))
        for h in hs:
            h.wait()


def _run_sc_gather(table, idxw):
    mesh = plsc.VectorSubcoreMesh(core_axis_name="c", subcore_axis_name="s")
    fn = functools.partial(
        pl.kernel,
        mesh=mesh,
        out_type=jax.ShapeDtypeStruct((NMID, BSC, RR), jnp.float32),
        scratch_types=[
            pltpu.VMEM((NMID, BPW), jnp.int32),
            pltpu.VMEM((HALF, BPW, RR), jnp.float32),
            pltpu.SemaphoreType.DMA,
            pltpu.SemaphoreType.DMA,
        ],
    )(_sc_body)
    return fn(table, idxw)


def _k2b_body(sT_ref, v0_ref, last_ref, nn_ref, out_ref):
    v = v0_ref[...]
    for d in range(NMID):
        sT = sT_ref[d]
        acc = v[0:1, :] * sT[0:R, :]
        for r in range(1, R):
            acc = acc + v[r:r + 1, :] * sT[r * R:(r + 1) * R, :]
        v = acc
    out_ref[...] = jnp.sum(v * last_ref[...], axis=0) + nn_ref[...]


def _run_k2b(sT, v0, last, nn):
    whole = lambda shape: pl.BlockSpec(shape, lambda: tuple(0 for _ in shape))
    return pl.pallas_call(
        _k2b_body,
        in_specs=[whole((NMID, RR, BSC)), whole((R, BSC)), whole((R, BSC)),
                  whole((BSC,))],
        out_specs=whole((BSC,)),
        out_shape=jax.ShapeDtypeStruct((BSC,), jnp.float32),
    )(sT, v0, last, nn)


def _k2a_body(pT_ref, nodesT_ref, cfT_ref, cmT_ref, clT_ref,
              w1_ref, b1_ref, w2_ref, b2_ref, w3_ref, b3_ref, out_ref):
    f32 = jnp.float32
    dn = (((1,), (0,)), ((), ()))
    iota_col = lax.broadcasted_iota(jnp.int32, (M, 1), 0)

    def onehot(d):
        amin = _first_argmin_lanes(pT_ref[d:d + 1, :], nodesT_ref[:, d:d + 1],
                                   iota_col)
        return (iota_col == amin).astype(f32)

    v = lax.dot_general(cfT_ref[...], onehot(0), dn, preferred_element_type=f32)
    for i in range(NMID):
        sT = lax.dot_general(cmT_ref[i], onehot(i + 1), dn,
                             preferred_element_type=f32)
        acc = v[0:1, :] * sT[0:R, :]
        for r in range(1, R):
            acc = acc + v[r:r + 1, :] * sT[r * R:(r + 1) * R, :]
        v = acc
    lastT = lax.dot_general(clT_ref[...], onehot(D - 1), dn,
                            preferred_element_type=f32)
    tt = jnp.sum(v * lastT, axis=0)
    nn = _mlp_lanes(pT_ref[...], w1_ref, b1_ref, w2_ref, b2_ref, w3_ref, b3_ref)
    out_ref[...] = tt + nn[0, :]


def _run_k2a(pT_main, nodesT, cfT, cmT, clT, w1, b1c, w2, b2c, w3, b3):
    whole = lambda shape: pl.BlockSpec(shape, lambda i: tuple(0 for _ in shape))
    return pl.pallas_call(
        _k2a_body,
        grid=(BMAIN // BBLK,),
        in_specs=[
            pl.BlockSpec((D, BBLK), lambda i: (0, i)),
            whole((M, D)),
            whole((R, M)),
            whole((NMID, RR, M)),
            whole((R, M)),
            whole((H, D)),
            whole((H, 1)),
            whole((H, H)),
            whole((H, 1)),
            whole((1, H)),
            whole((1,)),
        ],
        out_specs=pl.BlockSpec((BBLK,), lambda i: (i,)),
        out_shape=jax.ShapeDtypeStruct((BMAIN,), jnp.float32),
    )(pT_main, nodesT, cfT, cmT, clT, w1, b1c, w2, b2c, w3, b3)


@jax.jit
def kernel(points, core_first, cores_mid, core_last, nodes, W1, b1, W2, b2, W3, b3):
    pT = points.T
    nodesT = nodes.T
    cfT = core_first[0].T
    clT = core_last[:, :, 0]
    cmT = jnp.transpose(cores_mid, (0, 1, 3, 2)).reshape(NMID, RR, M)
    table = jnp.transpose(cores_mid, (0, 2, 1, 3)).reshape(NMID * M, RR)
    b1c, b2c = b1[:, None], b2[:, None]

    idx, v0, last, nn = _run_k1(pT[:, :BSC], nodesT, cfT, clT,
                                W1, b1c, W2, b2c, W3, b3)
    idxw = jnp.transpose(idx.reshape(NMID, NW, BPW), (1, 0, 2))
    S = _run_sc_gather(table, idxw)
    sT = jnp.transpose(S, (0, 2, 1))
    out_sc = _run_k2b(sT, v0, last, nn)

    out_main = _run_k2a(pT[:, BSC:], nodesT, cfT, cmT, clT,
                        W1, b1c, W2, b2c, W3, b3)
    return jnp.concatenate([out_sc, out_main])


# speedup vs baseline: 19.8321x; 1.1120x over previous
import functools

import jax
import jax.numpy as jnp
from jax import lax
from jax.experimental import pallas as pl
from jax.experimental.pallas import tpu as pltpu
from jax.experimental.pallas import tpu_sc as plsc

B = 16384
D = 26
M = 64
R = 16
RR = R * R
H = 52
NMID = D - 2
NW = 32
BSC = 1024
BPW = BSC // NW
HALF = NMID // 2
BMAIN = B - BSC
BBLK = 1024


def _first_argmin_lanes(x_row, nodes_col, iota_col):
    dist = jnp.abs(x_row - nodes_col)
    minv = jnp.min(dist, axis=0, keepdims=True)
    prio = jnp.where(dist == minv, iota_col, jnp.int32(M))
    return jnp.min(prio, axis=0, keepdims=True)


def _mlp_lanes(pT, w1_ref, b1_ref, w2_ref, b2_ref, w3_ref, b3_ref):
    f32 = jnp.float32
    dn = (((1,), (0,)), ((), ()))
    h1 = lax.dot_general(w1_ref[...], pT, dn, preferred_element_type=f32)
    h1 = jnp.maximum(h1 + b1_ref[...], 0.0)
    h2 = lax.dot_general(w2_ref[...], h1, dn, preferred_element_type=f32)
    h2 = jnp.maximum(h2 + b2_ref[...], 0.0)
    nn = lax.dot_general(w3_ref[...], h2, dn, preferred_element_type=f32)
    return nn + b3_ref[0]


def _k1_body(pT_ref, p_ref, nodesT_ref, nodes_ref, cf_ref, cl_ref,
             w1_ref, b1_ref, w2_ref, b2_ref, w3_ref, b3_ref,
             idx_ref, v0_ref, last_ref, nn_ref):
    f32 = jnp.float32
    dn = (((1,), (0,)), ((), ()))
    iota_col = lax.broadcasted_iota(jnp.int32, (M, 1), 0)
    rows = []
    for d in range(1, D - 1):
        amin = _first_argmin_lanes(pT_ref[d:d + 1, :], nodesT_ref[:, d:d + 1],
                                   iota_col)
        rows.append(amin + jnp.int32((d - 1) * M))
    idx_ref[...] = jnp.concatenate(rows, axis=0)

    iota_row = lax.broadcasted_iota(jnp.int32, (1, M), 1)

    def onehot_bs(d):
        dist = jnp.abs(p_ref[:, d:d + 1] - nodes_ref[d:d + 1, :])
        minv = jnp.min(dist, axis=1, keepdims=True)
        prio = jnp.where(dist == minv, iota_row, jnp.int32(M))
        amin = jnp.min(prio, axis=1, keepdims=True)
        return (iota_row == amin).astype(f32)

    v0_ref[...] = lax.dot_general(onehot_bs(0), cf_ref[...], dn,
                                  preferred_element_type=f32)
    last_ref[...] = lax.dot_general(onehot_bs(D - 1), cl_ref[...], dn,
                                    preferred_element_type=f32)
    nn_ref[...] = _mlp_lanes(pT_ref[...], w1_ref, b1_ref, w2_ref, b2_ref,
                             w3_ref, b3_ref)[0, :]


def _run_k1(pT_sc, p_sc, nodesT, nodes, cf, cl, w1, b1c, w2, b2c, w3, b3):
    whole = lambda shape: pl.BlockSpec(shape, lambda: tuple(0 for _ in shape))
    return pl.pallas_call(
        _k1_body,
        in_specs=[whole((D, BSC)), whole((BSC, D)), whole((M, D)),
                  whole((D, M)), whole((M, R)), whole((M, R)),
                  whole((H, D)), whole((H, 1)), whole((H, H)), whole((H, 1)),
                  whole((1, H)), whole((1,))],
        out_specs=[whole((NMID, BSC)), whole((BSC, R)), whole((BSC, R)),
                   whole((BSC,))],
        out_shape=[
            jax.ShapeDtypeStruct((NMID, BSC), jnp.int32),
            jax.ShapeDtypeStruct((BSC, R), jnp.float32),
            jax.ShapeDtypeStruct((BSC, R), jnp.float32),
            jax.ShapeDtypeStruct((BSC,), jnp.float32),
        ],
    )(pT_sc, p_sc, nodesT, nodes, cf, cl, w1, b1c, w2, b2c, w3, b3)


def _sc_body(table_hbm, idxw_hbm, out_hbm, idx_all, bufs, gsem, ssem):
    wid = lax.axis_index("s") * 2 + lax.axis_index("c")
    pltpu.sync_copy(idxw_hbm.at[wid], idx_all)
    for half in range(2):
        hg = []
        for t in range(HALF):
            d = half * HALF + t
            hg.append(pltpu.async_copy(table_hbm.at[idx_all.at[d]],
                                       bufs.at[t], gsem))
        for h in hg:
            h.wait()
        hs = []
        for t in range(HALF):
            d = half * HALF + t
            hs.append(pltpu.async_copy(
                bufs.at[t], out_hbm.at[d, pl.ds(wid * BPW, BPW)], ssem))
        for h in hs:
            h.wait()


def _run_sc_gather(table, idxw):
    mesh = plsc.VectorSubcoreMesh(core_axis_name="c", subcore_axis_name="s")
    fn = functools.partial(
        pl.kernel,
        mesh=mesh,
        out_type=jax.ShapeDtypeStruct((NMID, BSC, RR), jnp.float32),
        scratch_types=[
            pltpu.VMEM((NMID, BPW), jnp.int32),
            pltpu.VMEM((HALF, BPW, RR), jnp.float32),
            pltpu.SemaphoreType.DMA,
            pltpu.SemaphoreType.DMA,
        ],
    )(_sc_body)
    return fn(table, idxw)


def _k2b_body(s_ref, v0_ref, last_ref, nn_ref, e_ref, kt_ref, out_ref):
    f32 = jnp.float32
    dn = (((1,), (0,)), ((), ()))
    v = v0_ref[...]
    for d in range(NMID):
        vexp = lax.dot_general(v, e_ref[...], dn,
                               preferred_element_type=f32)
        v = lax.dot_general(vexp * s_ref[d], kt_ref[...], dn,
                            preferred_element_type=f32)
    out_ref[...] = jnp.sum(v * last_ref[...], axis=1) + nn_ref[...]


def _run_k2b(S, v0, last, nn, E, Kt):
    whole = lambda shape: pl.BlockSpec(shape, lambda: tuple(0 for _ in shape))
    return pl.pallas_call(
        _k2b_body,
        in_specs=[whole((NMID, BSC, RR)), whole((BSC, R)), whole((BSC, R)),
                  whole((BSC,)), whole((R, RR)), whole((RR, R))],
        out_specs=whole((BSC,)),
        out_shape=jax.ShapeDtypeStruct((BSC,), jnp.float32),
    )(S, v0, last, nn, E, Kt)


def _k2a_body(pT_ref, nodesT_ref, cfT_ref, cmT_ref, clT_ref,
              w1_ref, b1_ref, w2_ref, b2_ref, w3_ref, b3_ref, out_ref):
    f32 = jnp.float32
    dn = (((1,), (0,)), ((), ()))
    iota_col = lax.broadcasted_iota(jnp.int32, (M, 1), 0)

    def onehot(d):
        amin = _first_argmin_lanes(pT_ref[d:d + 1, :], nodesT_ref[:, d:d + 1],
                                   iota_col)
        return (iota_col == amin).astype(f32)

    v = lax.dot_general(cfT_ref[...], onehot(0), dn, preferred_element_type=f32)
    for i in range(NMID):
        sT = lax.dot_general(cmT_ref[i], onehot(i + 1), dn,
                             preferred_element_type=f32)
        acc = v[0:1, :] * sT[0:R, :]
        for r in range(1, R):
            acc = acc + v[r:r + 1, :] * sT[r * R:(r + 1) * R, :]
        v = acc
    lastT = lax.dot_general(clT_ref[...], onehot(D - 1), dn,
                            preferred_element_type=f32)
    tt = jnp.sum(v * lastT, axis=0)
    nn = _mlp_lanes(pT_ref[...], w1_ref, b1_ref, w2_ref, b2_ref, w3_ref, b3_ref)
    out_ref[...] = tt + nn[0, :]


def _run_k2a(pT_main, nodesT, cfT, cmT, clT, w1, b1c, w2, b2c, w3, b3):
    whole = lambda shape: pl.BlockSpec(shape, lambda i: tuple(0 for _ in shape))
    return pl.pallas_call(
        _k2a_body,
        grid=(BMAIN // BBLK,),
        in_specs=[
            pl.BlockSpec((D, BBLK), lambda i: (0, i)),
            whole((M, D)),
            whole((R, M)),
            whole((NMID, RR, M)),
            whole((R, M)),
            whole((H, D)),
            whole((H, 1)),
            whole((H, H)),
            whole((H, 1)),
            whole((1, H)),
            whole((1,)),
        ],
        out_specs=pl.BlockSpec((BBLK,), lambda i: (i,)),
        out_shape=jax.ShapeDtypeStruct((BMAIN,), jnp.float32),
    )(pT_main, nodesT, cfT, cmT, clT, w1, b1c, w2, b2c, w3, b3)


@jax.jit
def kernel(points, core_first, cores_mid, core_last, nodes, W1, b1, W2, b2, W3, b3):
    pT = points.T
    nodesT = nodes.T
    cfT = core_first[0].T
    clT = core_last[:, :, 0]
    cmT = jnp.transpose(cores_mid, (0, 1, 3, 2)).reshape(NMID, RR, M)
    table = jnp.transpose(cores_mid, (0, 2, 1, 3)).reshape(NMID * M, RR)
    b1c, b2c = b1[:, None], b2[:, None]

    ir = jnp.arange(R)
    irr = jnp.arange(RR)
    E = (irr[None, :] // R == ir[:, None]).astype(jnp.float32)
    Kt = (irr[:, None] % R == ir[None, :]).astype(jnp.float32)

    idx, v0, last, nn = _run_k1(pT[:, :BSC], points[:BSC], nodesT, nodes,
                                core_first[0], core_last[:, :, 0].T,
                                W1, b1c, W2, b2c, W3, b3)
    idxw = jnp.transpose(idx.reshape(NMID, NW, BPW), (1, 0, 2))
    S = _run_sc_gather(table, idxw)
    out_sc = _run_k2b(S, v0, last, nn, E, Kt)

    out_main = _run_k2a(pT[:, BSC:], nodesT, cfT, cmT, clT,
                        W1, b1c, W2, b2c, W3, b3)
    return jnp.concatenate([out_sc, out_main])
